# baseline (speedup 1.0000x reference)
import functools

import jax
import jax.numpy as jnp
from jax import lax
from jax.experimental import pallas as pl
from jax.experimental.pallas import tpu as pltpu
from jax.experimental.pallas import tpu_sc as plsc

NUM_FIELDS = 26
FIELD_SIZE = 100000
EMBED = 16
LANES = 16
NUM_WORKERS = 32
CHUNK = 128


def _make_kernel(n_rows: int):
    per_w = n_rows // NUM_WORKERS
    n_chunks = per_w // CHUNK
    mesh = plsc.VectorSubcoreMesh(core_axis_name="c", subcore_axis_name="s")

    @functools.partial(
        pl.kernel,
        out_type=jax.ShapeDtypeStruct((n_rows, EMBED), jnp.float32),
        mesh=mesh,
        compiler_params=pltpu.CompilerParams(use_tc_tiling_on_sc=False),
        scratch_types=[
            pltpu.VMEM((per_w,), jnp.int32),
            pltpu.VMEM((CHUNK, EMBED), jnp.float32),
            pltpu.SemaphoreType.DMA,
        ],
    )
    def run(x_hbm, table_hbm, out_hbm, idx_v, rows_v, gsem):
        wid = lax.axis_index("s") * 2 + lax.axis_index("c")
        base = wid * per_w
        pltpu.sync_copy(x_hbm.at[pl.ds(base, per_w)], idx_v)

        lane = lax.broadcasted_iota(jnp.int32, (LANES,), 0)

        def add_offsets(j, _):
            off = pl.multiple_of(j * LANES, LANES)
            pos = base + off + lane
            field = lax.rem(pos, NUM_FIELDS)
            idx_v[pl.ds(off, LANES)] = (
                idx_v[pl.ds(off, LANES)] + field * FIELD_SIZE
            )
            return 0

        lax.fori_loop(0, per_w // LANES, add_offsets, 0)

        def gather_chunk(j, _):
            off = pl.multiple_of(j * CHUNK, CHUNK)
            pltpu.async_copy(
                table_hbm.at[idx_v.at[pl.ds(off, CHUNK)]], rows_v, gsem
            ).wait()
            pltpu.sync_copy(rows_v, out_hbm.at[pl.ds(base + off, CHUNK)])
            return 0

        lax.fori_loop(0, n_chunks, gather_chunk, 0)

    return run


def kernel(x, table):
    batch, num_fields = x.shape
    n_rows = batch * num_fields
    x_flat = x.reshape(n_rows)
    out_flat = _make_kernel(n_rows)(x_flat, table)
    return out_flat.reshape(batch, num_fields, EMBED)


# speedup vs baseline: 1.0452x; 1.0452x over previous
import functools

import jax
import jax.numpy as jnp
from jax import lax
from jax.experimental import pallas as pl
from jax.experimental.pallas import tpu as pltpu
from jax.experimental.pallas import tpu_sc as plsc

NUM_FIELDS = 26
FIELD_SIZE = 100000
EMBED = 16
LANES = 16
NUM_WORKERS = 32
CHUNK = 128
K = 4
GSZ = K * CHUNK


def _make_kernel(n_rows: int):
    per_w = n_rows // NUM_WORKERS
    n_groups = per_w // GSZ
    pairs = n_groups // 2
    mesh = plsc.VectorSubcoreMesh(core_axis_name="c", subcore_axis_name="s")

    @functools.partial(
        pl.kernel,
        out_type=jax.ShapeDtypeStruct((n_rows, EMBED), jnp.float32),
        mesh=mesh,
        compiler_params=pltpu.CompilerParams(use_tc_tiling_on_sc=False),
        scratch_types=[
            pltpu.VMEM((per_w,), jnp.int32),
            pltpu.VMEM((K, CHUNK, EMBED), jnp.float32),
            pltpu.VMEM((K, CHUNK, EMBED), jnp.float32),
            pltpu.SemaphoreType.DMA,
            pltpu.SemaphoreType.DMA,
            pltpu.SemaphoreType.DMA,
            pltpu.SemaphoreType.DMA,
        ],
    )
    def run(x_hbm, table_hbm, out_hbm, idx_v, buf_a, buf_b,
            gsem_a, gsem_b, ssem_a, ssem_b):
        wid = lax.axis_index("s") * 2 + lax.axis_index("c")
        base = wid * per_w
        pltpu.sync_copy(x_hbm.at[pl.ds(base, per_w)], idx_v)

        lane = lax.broadcasted_iota(jnp.int32, (LANES,), 0)

        def prep(g):
            for v in range(GSZ // LANES):
                off = pl.multiple_of(g * GSZ + v * LANES, LANES)
                field = lax.rem(base + off + lane, NUM_FIELDS)
                idx_v[pl.ds(off, LANES)] = (
                    idx_v[pl.ds(off, LANES)] + field * FIELD_SIZE
                )

        def fire_gathers(g, buf, sem):
            for b in range(K):
                off = pl.multiple_of(g * GSZ + b * CHUNK, CHUNK)
                pltpu.async_copy(
                    table_hbm.at[idx_v.at[pl.ds(off, CHUNK)]], buf.at[b], sem
                )

        def fire_stores(g, buf, sem):
            for b in range(K):
                off = pl.multiple_of(g * GSZ + b * CHUNK, CHUNK)
                pltpu.async_copy(
                    buf.at[b], out_hbm.at[pl.ds(base + off, CHUNK)], sem
                )

        def drain(sem, n):
            for _ in range(n):
                pltpu.make_async_copy(
                    buf_a.at[0], out_hbm.at[pl.ds(base, CHUNK)], sem
                ).wait()

        prep(0)
        fire_gathers(0, buf_a, gsem_a)
        prep(1)
        fire_gathers(1, buf_b, gsem_b)
        drain(gsem_a, K)
        fire_stores(0, buf_a, ssem_a)

        def body(t, _):
            g0 = pl.multiple_of(2 * t, 2)
            g1 = g0 + 1
            prep(g0)
            drain(ssem_a, K)
            fire_gathers(g0, buf_a, gsem_a)
            drain(gsem_b, K)
            fire_stores(g1 - 2, buf_b, ssem_b)
            prep(g1)
            drain(ssem_b, K)
            fire_gathers(g1, buf_b, gsem_b)
            drain(gsem_a, K)
            fire_stores(g0, buf_a, ssem_a)
            return 0

        lax.fori_loop(1, pairs, body, 0)

        drain(gsem_b, K)
        fire_stores(n_groups - 1, buf_b, ssem_b)
        drain(ssem_a, K)
        drain(ssem_b, K)

    return run


def kernel(x, table):
    batch, num_fields = x.shape
    n_rows = batch * num_fields
    x_flat = x.reshape(n_rows)
    out_flat = _make_kernel(n_rows)(x_flat, table)
    return out_flat.reshape(batch, num_fields, EMBED)


# speedup vs baseline: 1.1184x; 1.0701x over previous
import functools

import jax
import jax.numpy as jnp
from jax import lax
from jax.experimental import pallas as pl
from jax.experimental.pallas import tpu as pltpu
from jax.experimental.pallas import tpu_sc as plsc

NUM_FIELDS = 26
FIELD_SIZE = 100000
EMBED = 16
LANES = 16
NUM_WORKERS = 32
CHUNK = 128


def _make_kernel(n_rows: int, n_table_rows: int):
    per_w = n_rows // NUM_WORKERS
    n_chunks = per_w // CHUNK
    pairs = n_chunks // 2
    out_rows = n_rows * EMBED // 128
    mesh = plsc.VectorSubcoreMesh(core_axis_name="c", subcore_axis_name="s")

    @functools.partial(
        pl.kernel,
        out_type=jax.ShapeDtypeStruct((out_rows, 128), jnp.float32),
        mesh=mesh,
        compiler_params=pltpu.CompilerParams(
            use_tc_tiling_on_sc=False, needs_layout_passes=False),
        scratch_types=[
            pltpu.VMEM((per_w,), jnp.int32),
            pltpu.VMEM((per_w,), jnp.int32),
            pltpu.VMEM((CHUNK, 8, EMBED), jnp.float32),
            pltpu.VMEM((CHUNK, 8, EMBED), jnp.float32),
            pltpu.VMEM((CHUNK * EMBED // 128, 128), jnp.float32),
            pltpu.VMEM((CHUNK * EMBED // 128, 128), jnp.float32),
            pltpu.SemaphoreType.DMA,
            pltpu.SemaphoreType.DMA,
            pltpu.SemaphoreType.DMA,
            pltpu.SemaphoreType.DMA,
        ],
    )
    def run(x_hbm, tbl_hbm, out_hbm, gidx, jidx, tiles_a, tiles_b,
            rows_a, rows_b, gsem_a, gsem_b, ssem_a, ssem_b):
        wid = lax.axis_index("s") * 2 + lax.axis_index("c")
        base = wid * per_w
        obase = base // 8
        orows = CHUNK * EMBED // 128
        pltpu.sync_copy(x_hbm.at[pl.ds(base, per_w)], gidx)

        lane = lax.broadcasted_iota(jnp.int32, (LANES,), 0)

        def prep(c):
            for v in range(CHUNK // LANES):
                off = pl.multiple_of(c * CHUNK + v * LANES, LANES)
                field = lax.rem(base + off + lane, NUM_FIELDS)
                r = gidx[pl.ds(off, LANES)] + field * FIELD_SIZE
                gidx[pl.ds(off, LANES)] = lax.shift_right_logical(r, 3)
                jidx[pl.ds(off, LANES)] = lax.bitwise_and(r, 7)

        def fire_gather(c, tiles, sem):
            off = pl.multiple_of(c * CHUNK, CHUNK)
            pltpu.async_copy(tbl_hbm.at[gidx.at[pl.ds(off, CHUNK)]], tiles, sem)

        def extract(c, tiles, rows):
            def blk(v, _):
                off = pl.multiple_of(c * CHUNK + v * LANES, LANES)
                jv = jidx[pl.ds(off, LANES)]
                rowv = lane + v * LANES
                pbase = rowv * EMBED
                for col in range(EMBED):
                    vals = plsc.load_gather(tiles, [rowv, jv, lane * 0 + col])
                    p = pbase + col
                    plsc.store_scatter(
                        rows,
                        [lax.shift_right_logical(p, 7), lax.bitwise_and(p, 127)],
                        vals,
                    )
                return 0

            lax.fori_loop(0, CHUNK // LANES, blk, 0)

        def fire_store(c, rows, sem):
            off = pl.multiple_of(obase + c * orows, orows)
            pltpu.async_copy(rows, out_hbm.at[pl.ds(off, orows)], sem)

        def drain_g(sem):
            pltpu.make_async_copy(
                tbl_hbm.at[gidx.at[pl.ds(0, CHUNK)]], tiles_a, sem
            ).wait()

        def drain_s(sem):
            pltpu.make_async_copy(
                rows_a, out_hbm.at[pl.ds(obase, orows)], sem
            ).wait()

        prep(0)
        fire_gather(0, tiles_a, gsem_a)
        prep(1)
        fire_gather(1, tiles_b, gsem_b)
        drain_g(gsem_a)
        extract(0, tiles_a, rows_a)
        fire_store(0, rows_a, ssem_a)
        prep(2)
        fire_gather(2, tiles_a, gsem_a)
        drain_g(gsem_b)
        extract(1, tiles_b, rows_b)
        fire_store(1, rows_b, ssem_b)
        prep(3)
        fire_gather(3, tiles_b, gsem_b)

        def body(t, _):
            c0 = pl.multiple_of(2 * t, 2)
            c1 = c0 + 1
            drain_g(gsem_a)
            drain_s(ssem_a)
            extract(c0, tiles_a, rows_a)
            fire_store(c0, rows_a, ssem_a)
            prep(c0 + 2)
            fire_gather(c0 + 2, tiles_a, gsem_a)
            drain_g(gsem_b)
            drain_s(ssem_b)
            extract(c1, tiles_b, rows_b)
            fire_store(c1, rows_b, ssem_b)
            prep(c1 + 2)
            fire_gather(c1 + 2, tiles_b, gsem_b)
            return 0

        lax.fori_loop(1, pairs - 1, body, 0)

        drain_g(gsem_a)
        drain_s(ssem_a)
        extract(n_chunks - 2, tiles_a, rows_a)
        fire_store(n_chunks - 2, rows_a, ssem_a)
        drain_g(gsem_b)
        drain_s(ssem_b)
        extract(n_chunks - 1, tiles_b, rows_b)
        fire_store(n_chunks - 1, rows_b, ssem_b)
        drain_s(ssem_a)
        drain_s(ssem_b)

    return run


def kernel(x, table):
    batch, num_fields = x.shape
    n_rows = batch * num_fields
    x_flat = x.reshape(n_rows)
    tbl3 = table.reshape(table.shape[0] // 8, 8, EMBED)
    out_flat = _make_kernel(n_rows, table.shape[0])(x_flat, tbl3)
    return out_flat.reshape(batch, num_fields, EMBED)


# speedup vs baseline: 1.2054x; 1.0777x over previous
import functools

import jax
import jax.numpy as jnp
from jax import lax
from jax.experimental import pallas as pl
from jax.experimental.pallas import tpu as pltpu
from jax.experimental.pallas import tpu_sc as plsc

NUM_FIELDS = 26
FIELD_SIZE = 100000
EMBED = 16
LANES = 16
NUM_WORKERS = 32
BROWS = 4
CHUNK = BROWS * NUM_FIELDS
K = 4
GSZ = K * CHUNK


def _make_kernel(batch: int, n_rows: int):
    per_w = n_rows // NUM_WORKERS
    n_groups = per_w // GSZ
    pairs = n_groups // 2
    b_per_w = batch // NUM_WORKERS
    mesh = plsc.VectorSubcoreMesh(core_axis_name="c", subcore_axis_name="s")

    @functools.partial(
        pl.kernel,
        out_type=jax.ShapeDtypeStruct((batch, NUM_FIELDS, EMBED), jnp.float32),
        mesh=mesh,
        compiler_params=pltpu.CompilerParams(
            use_tc_tiling_on_sc=False, needs_layout_passes=False),
        scratch_types=[
            pltpu.VMEM((per_w,), jnp.int32),
            pltpu.VMEM((K, CHUNK, EMBED), jnp.float32),
            pltpu.VMEM((K, CHUNK, EMBED), jnp.float32),
            pltpu.SemaphoreType.DMA,
            pltpu.SemaphoreType.DMA,
            pltpu.SemaphoreType.DMA,
            pltpu.SemaphoreType.DMA,
        ],
    )
    def run(x_hbm, table_hbm, out_hbm, idx_v, buf_a, buf_b,
            gsem_a, gsem_b, ssem_a, ssem_b):
        wid = lax.axis_index("s") * 2 + lax.axis_index("c")
        base = wid * per_w
        brow0 = wid * b_per_w
        pltpu.sync_copy(x_hbm.at[pl.ds(base, per_w)], idx_v)

        lane = lax.broadcasted_iota(jnp.int32, (LANES,), 0)

        def prep(g):
            for v in range(GSZ // LANES):
                off = pl.multiple_of(g * GSZ + v * LANES, LANES)
                field = lax.rem(base + off + lane, NUM_FIELDS)
                idx_v[pl.ds(off, LANES)] = (
                    idx_v[pl.ds(off, LANES)] + field * FIELD_SIZE
                )

        def fire_gathers(g, buf, sem):
            for b in range(K):
                off = pl.multiple_of(g * GSZ + b * CHUNK, 8)
                pltpu.async_copy(
                    table_hbm.at[idx_v.at[pl.ds(off, CHUNK)]], buf.at[b], sem
                )

        def fire_stores(g, buf, sem):
            for b in range(K):
                row = pl.multiple_of(brow0 + g * (K * BROWS) + b * BROWS, BROWS)
                for r in range(BROWS):
                    pltpu.async_copy(
                        buf.at[b, pl.ds(r * NUM_FIELDS, NUM_FIELDS)],
                        out_hbm.at[row + r], sem
                    )

        def drain_g(sem, n):
            for _ in range(n):
                pltpu.make_async_copy(
                    table_hbm.at[idx_v.at[pl.ds(0, CHUNK)]], buf_a.at[0], sem
                ).wait()

        def drain_s(sem, n):
            for _ in range(n * BROWS):
                pltpu.make_async_copy(
                    buf_a.at[0, pl.ds(0, NUM_FIELDS)], out_hbm.at[brow0], sem
                ).wait()

        prep(0)
        fire_gathers(0, buf_a, gsem_a)
        prep(1)
        fire_gathers(1, buf_b, gsem_b)
        drain_g(gsem_a, K)
        fire_stores(0, buf_a, ssem_a)

        def body(t, _):
            g0 = pl.multiple_of(2 * t, 2)
            g1 = g0 + 1
            prep(g0)
            drain_s(ssem_a, K)
            fire_gathers(g0, buf_a, gsem_a)
            drain_g(gsem_b, K)
            fire_stores(g1 - 2, buf_b, ssem_b)
            prep(g1)
            drain_s(ssem_b, K)
            fire_gathers(g1, buf_b, gsem_b)
            drain_g(gsem_a, K)
            fire_stores(g0, buf_a, ssem_a)
            return 0

        lax.fori_loop(1, pairs, body, 0)

        drain_g(gsem_b, K)
        fire_stores(n_groups - 1, buf_b, ssem_b)
        drain_s(ssem_a, K)
        drain_s(ssem_b, K)

    return run


def kernel(x, table):
    batch, num_fields = x.shape
    n_rows = batch * num_fields
    x_flat = x.reshape(n_rows)
    return _make_kernel(batch, n_rows)(x_flat, table)
